# baseline (speedup 1.0000x reference)
import functools

import jax
import jax.numpy as jnp
from jax import lax
from jax.experimental import pallas as pl
from jax.experimental.pallas import tpu as pltpu
from jax.experimental.pallas import tpu_sc as plsc

N = 10000
E = 320000
D = 128
M = 32
AVG_DEG = 32.0

NC = 2
NS = 16
NW = NC * NS
CH = 128
K = -(-E // (NW * CH))
E_PAD = NW * K * CH
TW = 128
N_ACC = 10112
RPT = N_ACC // NS

_mesh = plsc.VectorSubcoreMesh(
    core_axis_name="c", subcore_axis_name="s", num_cores=NC, num_subcores=NS)


@functools.partial(
    pl.kernel,
    out_type=jax.ShapeDtypeStruct((E_PAD, TW), jnp.float32),
    mesh=_mesh,
    scratch_types=[
        pltpu.VMEM((K, CH), jnp.int32),
        pltpu.VMEM((K, CH), jnp.int32),
        pltpu.VMEM((CH, TW), jnp.float32),
        pltpu.SemaphoreType.DMA,
        pltpu.SemaphoreType.DMA,
    ],
)
def _sc_gather(a_hbm, b_hbm, srcs_hbm, dsts_hbm, g_hbm,
               idx_a, idx_b, buf, sema, semb):
    c = lax.axis_index("c")
    s = lax.axis_index("s")
    wid = s * NC + c
    base = wid * (K * CH)
    pltpu.sync_copy(srcs_hbm.at[wid], idx_a)
    pltpu.sync_copy(dsts_hbm.at[wid], idx_b)

    @pl.loop(0, K)
    def _chunk(cc):
        off = base + cc * CH
        pltpu.async_copy(a_hbm.at[idx_a.at[cc]], buf, sema).wait()
        pltpu.async_copy(b_hbm.at[idx_b.at[cc]], buf, semb, add=True).wait()
        pltpu.sync_copy(buf, g_hbm.at[pl.ds(off, CH)])


@functools.partial(
    pl.kernel,
    out_type=jax.ShapeDtypeStruct((NC * N_ACC, TW), jnp.float32),
    mesh=_mesh,
    scratch_types=[
        pltpu.VMEM((K, CH), jnp.int32),
        pltpu.VMEM((CH, TW), jnp.float32),
        pltpu.VMEM_SHARED((N_ACC, TW), jnp.float32),
    ],
)
def _sc_scatter(s_hbm, dsts_hbm, z_hbm, out_hbm, idx, sbuf, accum):
    c = lax.axis_index("c")
    s = lax.axis_index("s")
    wid = s * NC + c
    base = wid * (K * CH)
    pltpu.sync_copy(z_hbm.at[pl.ds(s * RPT, RPT)], accum.at[pl.ds(s * RPT, RPT)])
    pltpu.sync_copy(dsts_hbm.at[wid], idx)
    plsc.subcore_barrier()

    @pl.loop(0, K)
    def _chunk(cc):
        pltpu.sync_copy(s_hbm.at[pl.ds(base + cc * CH, CH)], sbuf)
        pltpu.sync_copy(sbuf, accum.at[idx.at[cc]], add=True)

    plsc.subcore_barrier()
    pltpu.sync_copy(accum.at[pl.ds(s * RPT, RPT)],
                    out_hbm.at[pl.ds(c * N_ACC + s * RPT, RPT)])


def _tables_body(h_ref, c_ref, w1a_ref, w1b_ref, b1_ref, a_ref, b_ref):
    h = h_ref[...]
    pa = jnp.dot(h, w1a_ref[...], preferred_element_type=jnp.float32) + b1_ref[...]
    pb = jnp.dot(h, w1b_ref[...], preferred_element_type=jnp.float32)
    coords = c_ref[...]
    pad = jnp.zeros((h.shape[0], TW - M - 3), jnp.float32)
    a_ref[...] = jnp.concatenate([pa, coords, pad], axis=1)
    b_ref[...] = jnp.concatenate([pb, -coords, pad], axis=1)


def _edge_body(s1_ref, wd_ref, we2_ref, be2_ref, wc1_ref, bc1_ref, wc2_ref,
               emb_ref, msk_ref, g_ref, s_ref):
    g = g_ref[...]
    m1in = (jnp.dot(g, s1_ref[...], preferred_element_type=jnp.float32)
            + jnp.dot(g * g, wd_ref[...], preferred_element_type=jnp.float32))
    m = jax.nn.silu(m1in)
    m = jax.nn.silu(jnp.dot(m, we2_ref[...], preferred_element_type=jnp.float32)
                    + be2_ref[...])
    t = jax.nn.silu(jnp.dot(m, wc1_ref[...], preferred_element_type=jnp.float32)
                    + bc1_ref[...])
    cw = jnp.tanh(jnp.dot(t, wc2_ref[...], preferred_element_type=jnp.float32))
    s_ref[...] = (jnp.dot(m, emb_ref[...], preferred_element_type=jnp.float32)
                  + g * (cw * msk_ref[...]))


def _node_body(c_ref, h_ref, parts_ref, wn1a_ref, wn1b_ref, bn1_ref,
               wn2_ref, bn2_ref, oc_ref, oh_ref):
    parts = parts_ref[...]
    agg = parts[:N, :] + parts[N_ACC:N_ACC + N, :]
    agg_m = agg[:, :M]
    agg_c = agg[:, M:M + 3]
    oc_ref[...] = c_ref[...] + agg_c * (1.0 / AVG_DEG)
    h = h_ref[...]
    u = jax.nn.silu(
        jnp.dot(h, wn1a_ref[...], preferred_element_type=jnp.float32)
        + jnp.dot(agg_m, wn1b_ref[...], preferred_element_type=jnp.float32)
        + bn1_ref[...])
    oh = h + jnp.dot(u, wn2_ref[...], preferred_element_type=jnp.float32) + bn2_ref[...]
    hc = oh - jnp.mean(oh, axis=0, keepdims=True)
    denom = jnp.sqrt(jnp.mean(jnp.sum(hc * hc, axis=1)) + 1e-6)
    oh_ref[...] = hc / denom


def kernel(coords, hidden, edges, W_e1, b_e1, W_e2, b_e2, W_c1, b_c1, W_c2,
           W_n1, b_n1, W_n2, b_n2):
    src = edges[0].astype(jnp.int32)
    dst = edges[1].astype(jnp.int32)
    pad = E_PAD - E
    src_g = jnp.concatenate([src, jnp.zeros((pad,), jnp.int32)]).reshape(NW, K, CH)
    dst_g = jnp.concatenate([dst, jnp.zeros((pad,), jnp.int32)]).reshape(NW, K, CH)
    dst_s = jnp.concatenate([dst, jnp.full((pad,), N, jnp.int32)]).reshape(NW, K, CH)

    w1a = W_e1[:D]
    w1b = W_e1[D:2 * D]
    w1c = W_e1[2 * D]

    eye_m = jnp.eye(M, dtype=jnp.float32)
    s1 = jnp.zeros((TW, M), jnp.float32).at[:M, :].set(eye_m)
    wd = jnp.zeros((TW, M), jnp.float32).at[M:M + 3, :].set(
        jnp.broadcast_to(w1c, (3, M)))
    emb = jnp.zeros((M, TW), jnp.float32).at[:, :M].set(eye_m)
    msk = jnp.zeros((1, TW), jnp.float32).at[0, M:M + 3].set(1.0)

    tab_a, tab_b = pl.pallas_call(
        _tables_body,
        out_shape=[jax.ShapeDtypeStruct((N, TW), jnp.float32),
                   jax.ShapeDtypeStruct((N, TW), jnp.float32)],
    )(hidden, coords, w1a, w1b, b_e1.reshape(1, M))

    g = _sc_gather(tab_a, tab_b, src_g, dst_g)

    BE = 2048
    n_blk = E_PAD // BE
    s_rows = pl.pallas_call(
        _edge_body,
        grid=(n_blk,),
        in_specs=[
            pl.BlockSpec((TW, M), lambda i: (0, 0)),
            pl.BlockSpec((TW, M), lambda i: (0, 0)),
            pl.BlockSpec((M, M), lambda i: (0, 0)),
            pl.BlockSpec((1, M), lambda i: (0, 0)),
            pl.BlockSpec((M, M), lambda i: (0, 0)),
            pl.BlockSpec((1, M), lambda i: (0, 0)),
            pl.BlockSpec((M, 1), lambda i: (0, 0)),
            pl.BlockSpec((M, TW), lambda i: (0, 0)),
            pl.BlockSpec((1, TW), lambda i: (0, 0)),
            pl.BlockSpec((BE, TW), lambda i: (i, 0)),
        ],
        out_specs=pl.BlockSpec((BE, TW), lambda i: (i, 0)),
        out_shape=jax.ShapeDtypeStruct((E_PAD, TW), jnp.float32),
    )(s1, wd, W_e2, b_e2.reshape(1, M), W_c1, b_c1.reshape(1, M), W_c2,
      emb, msk, g)

    zeros_acc = jnp.zeros((N_ACC, TW), jnp.float32)
    parts = _sc_scatter(s_rows, dst_s, zeros_acc)

    out_coords, out_hidden = pl.pallas_call(
        _node_body,
        out_shape=[jax.ShapeDtypeStruct((N, 3), jnp.float32),
                   jax.ShapeDtypeStruct((N, D), jnp.float32)],
    )(coords, hidden, parts, W_n1[:D], W_n1[D:], b_n1.reshape(1, D),
      W_n2, b_n2.reshape(1, D))

    return out_coords, out_hidden


# speedup vs baseline: 1.4907x; 1.4907x over previous
import functools

import jax
import jax.numpy as jnp
from jax import lax
from jax.experimental import pallas as pl
from jax.experimental.pallas import tpu as pltpu
from jax.experimental.pallas import tpu_sc as plsc

N = 10000
E = 320000
D = 128
M = 32
AVG_DEG = 32.0

NC = 2
NS = 16
NW = NC * NS
CH = 128
K = 80
E_PAD = NW * K * CH
TW = 64
N_ACC = 10112
RPT = N_ACC // NS

_mesh = plsc.VectorSubcoreMesh(
    core_axis_name="c", subcore_axis_name="s", num_cores=NC, num_subcores=NS)
_sc_params = pltpu.CompilerParams(use_tc_tiling_on_sc=False)


@functools.partial(
    pl.kernel,
    out_type=jax.ShapeDtypeStruct((E_PAD, TW), jnp.float32),
    mesh=_mesh,
    scratch_types=[
        pltpu.VMEM((K, CH), jnp.int32),
        pltpu.VMEM((K, CH), jnp.int32),
        pltpu.VMEM((CH, TW), jnp.float32),
        pltpu.VMEM((CH, TW), jnp.float32),
        pltpu.SemaphoreType.DMA,
        pltpu.SemaphoreType.DMA,
        pltpu.SemaphoreType.DMA,
        pltpu.SemaphoreType.DMA,
        pltpu.SemaphoreType.DMA,
        pltpu.SemaphoreType.DMA,
    ],
    compiler_params=_sc_params,
)
def _sc_gather(a_hbm, b_hbm, srcs_hbm, dsts_hbm, g_hbm,
               idx_a, idx_b, buf0, buf1, sa0, sb0, sw0, sa1, sb1, sw1):
    c = lax.axis_index("c")
    s = lax.axis_index("s")
    wid = s * NC + c
    base = wid * (K * CH)
    pltpu.sync_copy(srcs_hbm.at[wid], idx_a)
    pltpu.sync_copy(dsts_hbm.at[wid], idx_b)

    @pl.loop(0, K // 2)
    def _step(st):
        cc0 = st * 2
        cc1 = cc0 + 1
        a0 = pltpu.async_copy(a_hbm.at[idx_a.at[cc0]], buf0, sa0)
        a1 = pltpu.async_copy(a_hbm.at[idx_a.at[cc1]], buf1, sa1)
        a0.wait()
        b0 = pltpu.async_copy(b_hbm.at[idx_b.at[cc0]], buf0, sb0, add=True)
        a1.wait()
        b1 = pltpu.async_copy(b_hbm.at[idx_b.at[cc1]], buf1, sb1, add=True)
        b0.wait()
        w0 = pltpu.async_copy(buf0, g_hbm.at[pl.ds(base + cc0 * CH, CH)], sw0)
        b1.wait()
        w1 = pltpu.async_copy(buf1, g_hbm.at[pl.ds(base + cc1 * CH, CH)], sw1)
        w0.wait()
        w1.wait()


@functools.partial(
    pl.kernel,
    out_type=jax.ShapeDtypeStruct((NC * N_ACC, TW), jnp.float32),
    mesh=_mesh,
    scratch_types=[
        pltpu.VMEM((K, CH), jnp.int32),
        pltpu.VMEM((CH, TW), jnp.float32),
        pltpu.VMEM((CH, TW), jnp.float32),
        pltpu.VMEM_SHARED((N_ACC, TW), jnp.float32),
        pltpu.SemaphoreType.DMA,
        pltpu.SemaphoreType.DMA,
        pltpu.SemaphoreType.DMA,
        pltpu.SemaphoreType.DMA,
    ],
    compiler_params=_sc_params,
)
def _sc_scatter(s_hbm, dsts_hbm, z_hbm, out_hbm,
                idx, sbuf0, sbuf1, accum, sl0, sl1, sc0, sc1):
    c = lax.axis_index("c")
    s = lax.axis_index("s")
    wid = s * NC + c
    base = wid * (K * CH)
    pltpu.sync_copy(z_hbm.at[pl.ds(s * RPT, RPT)], accum.at[pl.ds(s * RPT, RPT)])
    pltpu.sync_copy(dsts_hbm.at[wid], idx)
    plsc.subcore_barrier()

    @pl.loop(0, K // 2)
    def _step(st):
        cc0 = st * 2
        cc1 = cc0 + 1
        l0 = pltpu.async_copy(s_hbm.at[pl.ds(base + cc0 * CH, CH)], sbuf0, sl0)
        l1 = pltpu.async_copy(s_hbm.at[pl.ds(base + cc1 * CH, CH)], sbuf1, sl1)
        l0.wait()
        a0 = pltpu.async_copy(sbuf0, accum.at[idx.at[cc0]], sc0, add=True)
        l1.wait()
        a1 = pltpu.async_copy(sbuf1, accum.at[idx.at[cc1]], sc1, add=True)
        a0.wait()
        a1.wait()

    plsc.subcore_barrier()
    pltpu.sync_copy(accum.at[pl.ds(s * RPT, RPT)],
                    out_hbm.at[pl.ds(c * N_ACC + s * RPT, RPT)])


def _tables_body(h_ref, c_ref, w1a_ref, w1b_ref, b1_ref, a_ref, b_ref):
    h = h_ref[...]
    pa = jnp.dot(h, w1a_ref[...], preferred_element_type=jnp.float32) + b1_ref[...]
    pb = jnp.dot(h, w1b_ref[...], preferred_element_type=jnp.float32)
    coords = c_ref[...]
    pad = jnp.zeros((h.shape[0], TW - M - 3), jnp.float32)
    a_ref[...] = jnp.concatenate([pa, coords, pad], axis=1)
    b_ref[...] = jnp.concatenate([pb, -coords, pad], axis=1)


def _edge_body(s1_ref, wd_ref, we2_ref, be2_ref, wc1_ref, bc1_ref, wc2_ref,
               emb_ref, msk_ref, g_ref, s_ref):
    g = g_ref[...]
    m1in = (jnp.dot(g, s1_ref[...], preferred_element_type=jnp.float32)
            + jnp.dot(g * g, wd_ref[...], preferred_element_type=jnp.float32))
    m = jax.nn.silu(m1in)
    m = jax.nn.silu(jnp.dot(m, we2_ref[...], preferred_element_type=jnp.float32)
                    + be2_ref[...])
    t = jax.nn.silu(jnp.dot(m, wc1_ref[...], preferred_element_type=jnp.float32)
                    + bc1_ref[...])
    cw = jnp.tanh(jnp.dot(t, wc2_ref[...], preferred_element_type=jnp.float32))
    s_ref[...] = (jnp.dot(m, emb_ref[...], preferred_element_type=jnp.float32)
                  + g * jnp.dot(cw, msk_ref[...],
                                preferred_element_type=jnp.float32))


def _node_body(c_ref, h_ref, parts_ref, wn1a_ref, wn1b_ref, bn1_ref,
               wn2_ref, bn2_ref, oc_ref, oh_ref):
    parts = parts_ref[...]
    agg = parts[:N, :] + parts[N_ACC:N_ACC + N, :]
    agg_m = agg[:, :M]
    agg_c = agg[:, M:M + 3]
    oc_ref[...] = c_ref[...] + agg_c * (1.0 / AVG_DEG)
    h = h_ref[...]
    u = jax.nn.silu(
        jnp.dot(h, wn1a_ref[...], preferred_element_type=jnp.float32)
        + jnp.dot(agg_m, wn1b_ref[...], preferred_element_type=jnp.float32)
        + bn1_ref[...])
    oh = h + jnp.dot(u, wn2_ref[...], preferred_element_type=jnp.float32) + bn2_ref[...]
    hc = oh - jnp.mean(oh, axis=0, keepdims=True)
    denom = jnp.sqrt(jnp.mean(jnp.sum(hc * hc, axis=1)) + 1e-6)
    oh_ref[...] = hc / denom


def _blockdiag(w):
    r, c = w.shape
    z = jnp.zeros((r, c), jnp.float32)
    return jnp.concatenate([jnp.concatenate([w, z], axis=1),
                            jnp.concatenate([z, w], axis=1)], axis=0)


def kernel(coords, hidden, edges, W_e1, b_e1, W_e2, b_e2, W_c1, b_c1, W_c2,
           W_n1, b_n1, W_n2, b_n2):
    src = edges[0].astype(jnp.int32)
    dst = edges[1].astype(jnp.int32)
    pad = E_PAD - E
    src_g = jnp.concatenate([src, jnp.zeros((pad,), jnp.int32)]).reshape(NW, K, CH)
    dst_g = jnp.concatenate([dst, jnp.zeros((pad,), jnp.int32)]).reshape(NW, K, CH)
    dst_s = jnp.concatenate([dst, jnp.full((pad,), N, jnp.int32)]).reshape(NW, K, CH)

    w1a = W_e1[:D]
    w1b = W_e1[D:2 * D]
    w1c = W_e1[2 * D]

    eye_m = jnp.eye(M, dtype=jnp.float32)
    s1_h = jnp.zeros((TW, M), jnp.float32).at[:M, :].set(eye_m)
    wd_h = jnp.zeros((TW, M), jnp.float32).at[M:M + 3, :].set(
        jnp.broadcast_to(w1c, (3, M)))
    emb_h = jnp.zeros((M, TW), jnp.float32).at[:, :M].set(eye_m)
    msk_h = jnp.zeros((1, TW), jnp.float32).at[0, M:M + 3].set(1.0)

    s1_d = _blockdiag(s1_h)
    wd_d = _blockdiag(wd_h)
    we2_d = _blockdiag(W_e2)
    be2_d = jnp.tile(b_e2, 2).reshape(1, 2 * M)
    wc1_d = _blockdiag(W_c1)
    bc1_d = jnp.tile(b_c1, 2).reshape(1, 2 * M)
    wc2_d = _blockdiag(W_c2)
    emb_d = _blockdiag(emb_h)
    msk_d = _blockdiag(msk_h)

    tab_a, tab_b = pl.pallas_call(
        _tables_body,
        out_shape=[jax.ShapeDtypeStruct((N, TW), jnp.float32),
                   jax.ShapeDtypeStruct((N, TW), jnp.float32)],
    )(hidden, coords, w1a, w1b, b_e1.reshape(1, M))

    g = _sc_gather(tab_a, tab_b, src_g, dst_g)

    g2 = g.reshape(E_PAD // 2, 2 * TW)
    BE2 = 2048
    n_blk = E_PAD // 2 // BE2
    s2 = pl.pallas_call(
        _edge_body,
        grid=(n_blk,),
        in_specs=[
            pl.BlockSpec((2 * TW, 2 * M), lambda i: (0, 0)),
            pl.BlockSpec((2 * TW, 2 * M), lambda i: (0, 0)),
            pl.BlockSpec((2 * M, 2 * M), lambda i: (0, 0)),
            pl.BlockSpec((1, 2 * M), lambda i: (0, 0)),
            pl.BlockSpec((2 * M, 2 * M), lambda i: (0, 0)),
            pl.BlockSpec((1, 2 * M), lambda i: (0, 0)),
            pl.BlockSpec((2 * M, 2), lambda i: (0, 0)),
            pl.BlockSpec((2 * M, 2 * TW), lambda i: (0, 0)),
            pl.BlockSpec((2, 2 * TW), lambda i: (0, 0)),
            pl.BlockSpec((BE2, 2 * TW), lambda i: (i, 0)),
        ],
        out_specs=pl.BlockSpec((BE2, 2 * TW), lambda i: (i, 0)),
        out_shape=jax.ShapeDtypeStruct((E_PAD // 2, 2 * TW), jnp.float32),
    )(s1_d, wd_d, we2_d, be2_d, wc1_d, bc1_d, wc2_d, emb_d, msk_d, g2)

    s_rows = s2.reshape(E_PAD, TW)
    zeros_acc = jnp.zeros((N_ACC, TW), jnp.float32)
    parts = _sc_scatter(s_rows, dst_s, zeros_acc)

    out_coords, out_hidden = pl.pallas_call(
        _node_body,
        out_shape=[jax.ShapeDtypeStruct((N, 3), jnp.float32),
                   jax.ShapeDtypeStruct((N, D), jnp.float32)],
    )(coords, hidden, parts, W_n1[:D], W_n1[D:], b_n1.reshape(1, D),
      W_n2, b_n2.reshape(1, D))

    return out_coords, out_hidden
